# baseline (speedup 1.0000x reference)
import functools

import jax
import jax.numpy as jnp
from jax import lax
from jax.experimental import pallas as pl
from jax.experimental.pallas import tpu as pltpu
from jax.experimental.pallas import tpu_sc as plsc

B, L, V = 64, 8192, 21
CUTOFF = 0.1
NEG_INF = float("-inf")
LANES = 16
VECS = L // LANES
ROWS_PER_W = 2
CHUNK = 128
NCHUNK = L // CHUNK


def _tec_kernel(ent_hbm, loss_hbm, aa_hbm, wt_hbm, out_hbm,
                wt_v, loss_v, aa_v, idx_v, gat_v, out_v, sem, gsem):
    nc = 2
    wid = lax.axis_index("s") * nc + lax.axis_index("c")
    lane = lax.iota(jnp.int32, LANES)

    pltpu.sync_copy(wt_hbm, wt_v)

    def _zero(j, _):
        out_v[pl.ds(j * LANES, LANES)] = jnp.zeros((LANES,), jnp.float32)
        return 0
    lax.fori_loop(0, VECS, _zero, 0)

    VPC = CHUNK // LANES

    def do_row(r, _):
        b = wid * ROWS_PER_W + r
        pltpu.sync_copy(loss_hbm.at[b], loss_v)
        pltpu.sync_copy(aa_hbm.at[b], aa_v)

        base = b * (L * V)

        def _chunk_cp(c):
            return pltpu.make_async_copy(
                ent_hbm.at[idx_v.at[pl.ds(c * CHUNK, CHUNK)]],
                gat_v.at[pl.ds(c * CHUNK, CHUNK)], gsem)

        def _mkidx(c, cur):
            l0 = c * CHUNK
            for k in range(VPC):
                wt = wt_v[pl.ds(l0 + k * LANES, LANES)]
                idx_v[pl.ds(l0 + k * LANES, LANES)] = cur + (k * LANES * V) + wt
            _chunk_cp(c).start()
            return cur + CHUNK * V
        lax.fori_loop(0, NCHUNK, _mkidx, base + lane * V)

        def _drain(c, _):
            _chunk_cp(c).wait()
            return 0
        lax.fori_loop(0, NCHUNK, _drain, 0)

        ninf = jnp.full((LANES,), NEG_INF, jnp.float32)
        zero_i = jnp.zeros((LANES,), jnp.int32)

        def _insert(chain, s, iv):
            m1, m2, m3, i1, i2, i3 = chain
            g1 = s > m1
            n1 = jnp.where(g1, s, m1)
            d1 = jnp.where(g1, m1, s)
            j1 = jnp.where(g1, iv, i1)
            e1 = jnp.where(g1, i1, iv)

            g2 = d1 > m2
            n2 = jnp.where(g2, d1, m2)
            d2 = jnp.where(g2, m2, d1)
            j2 = jnp.where(g2, e1, i2)
            e2 = jnp.where(g2, i2, e1)

            g3 = d2 > m3
            n3 = jnp.where(g3, d2, m3)
            j3 = jnp.where(g3, e2, i3)
            return (n1, n2, n3, j1, j2, j3)

        def _tour(c, carry):
            ca, cb = carry
            l0 = c * CHUNK
            for k in range(VPC):
                o = k * LANES
                s = loss_v[pl.ds(l0 + o, LANES)] - gat_v[pl.ds(l0 + o, LANES)]
                mut = aa_v[pl.ds(l0 + o, LANES)] != wt_v[pl.ds(l0 + o, LANES)]
                s = jnp.where(mut, s, ninf)
                iv = l0 + o + lane
                if k % 2 == 0:
                    ca = _insert(ca, s, iv)
                else:
                    cb = _insert(cb, s, iv)
            return ca, cb

        chain0 = (ninf, ninf, ninf, zero_i, zero_i, zero_i)
        ca, cb = lax.fori_loop(0, NCHUNK, _tour, (chain0, chain0))

        (a1, a2, a3, ai1, ai2, ai3) = ca
        (b1, b2, b3, bi1, bi2, bi3) = cb
        c1 = a1 > b3
        c2 = a2 > b2
        c3 = a3 > b1
        m1 = jnp.where(c1, a1, b3)
        m2 = jnp.where(c2, a2, b2)
        m3 = jnp.where(c3, a3, b1)
        i1 = jnp.where(c1, ai1, bi3)
        i2 = jnp.where(c2, ai2, bi2)
        i3 = jnp.where(c3, ai3, bi1)

        s1, j1 = plsc.sort_key_val(m1, i1)
        s2, j2 = plsc.sort_key_val(m2, i2)
        s3, j3 = plsc.sort_key_val(m3, i3)

        r2 = lax.rev(s2, (0,))
        rj2 = lax.rev(j2, (0,))
        c = s1 >= r2
        t = jnp.where(c, s1, r2)
        tj = jnp.where(c, j1, rj2)
        t, tj = plsc.sort_key_val(t, tj)

        r3 = lax.rev(s3, (0,))
        rj3 = lax.rev(j3, (0,))
        c = t >= r3
        u = jnp.where(c, t, r3)
        uj = jnp.where(c, tj, rj3)
        u, uj = plsc.sort_key_val(u, uj)

        keep = (lane >= LANES - 3) & (u > CUTOFF)
        sig = 1.0 / (1.0 + jnp.exp(-jnp.where(keep, u, 0.0)))

        plsc.store_scatter(out_v, [uj], sig, mask=keep)
        pltpu.sync_copy(out_v, out_hbm.at[b])
        plsc.store_scatter(out_v, [uj], jnp.zeros((LANES,), jnp.float32),
                           mask=keep)
        return 0

    lax.fori_loop(0, ROWS_PER_W, do_row, 0)


@jax.jit
def _revor_sc(ent_flat, loss, aa_tensor, aa_wt):
    mesh = plsc.VectorSubcoreMesh(core_axis_name="c", subcore_axis_name="s")
    f = pl.kernel(
        _tec_kernel,
        mesh=mesh,
        out_type=jax.ShapeDtypeStruct((B, L), jnp.float32),
        scratch_types=[
            pltpu.VMEM((L,), jnp.int32),
            pltpu.VMEM((L,), jnp.float32),
            pltpu.VMEM((L,), jnp.int32),
            pltpu.VMEM((L,), jnp.int32),
            pltpu.VMEM((L,), jnp.float32),
            pltpu.VMEM((L,), jnp.float32),
            pltpu.SemaphoreType.DMA,
            pltpu.SemaphoreType.DMA,
        ],
        compiler_params=pltpu.CompilerParams(needs_layout_passes=False),
    )
    return f(ent_flat, loss, aa_tensor, aa_wt)


def kernel(entropy, loss, aa_tensor, aa_wt, max_step):
    del max_step
    ent_flat = entropy.reshape(B * L * V)
    return _revor_sc(ent_flat, loss, aa_tensor, aa_wt)


# speedup vs baseline: 6.1099x; 6.1099x over previous
import functools

import jax
import jax.numpy as jnp
from jax import lax
from jax.experimental import pallas as pl
from jax.experimental.pallas import tpu as pltpu
from jax.experimental.pallas import tpu_sc as plsc

B, L, V = 64, 8192, 21
CUTOFF = 0.1
NEG_INF = float("-inf")
LANES = 16
VECS = L // LANES
ROWS_PER_W = 2
CHUNK = 128
NCHUNK = L // CHUNK


def _tec_kernel(ent_hbm, loss_hbm, aa_hbm, wt_hbm, out_hbm,
                wt_v, loss_v, aa_v, idx_v, gat_v, out_v, sem, gsem):
    nc = 2
    wid = lax.axis_index("s") * nc + lax.axis_index("c")
    lane = lax.iota(jnp.int32, LANES)

    pltpu.sync_copy(wt_hbm, wt_v)

    def _zero(j, _):
        out_v[pl.ds(j * LANES, LANES)] = jnp.zeros((LANES,), jnp.float32)
        wt_v[pl.ds(j * LANES, LANES)] = wt_v[pl.ds(j * LANES, LANES)] << 19
        return 0
    lax.fori_loop(0, VECS, _zero, 0)

    VPC = CHUNK // LANES

    def do_row(r, _):
        b = wid * ROWS_PER_W + r
        pltpu.sync_copy(loss_hbm.at[b], loss_v)
        pltpu.sync_copy(aa_hbm.at[b], aa_v)

        base = (b // 8) * 65536 + (b % 8) * 128

        def _chunk_cp(c):
            return pltpu.make_async_copy(
                ent_hbm.at[idx_v.at[pl.ds(c * CHUNK, CHUNK)]],
                gat_v.at[pl.ds(c * CHUNK, CHUNK)], gsem)

        def _mkidx(c, cur):
            l0 = c * CHUNK
            for k in range(VPC):
                wts = wt_v[pl.ds(l0 + k * LANES, LANES)]
                idx_v[pl.ds(l0 + k * LANES, LANES)] = cur + (k * LANES) + wts
            _chunk_cp(c).start()
            return cur + 1024
        lax.fori_loop(0, NCHUNK, _mkidx, base + lane)

        def _drain(c, _):
            _chunk_cp(c).wait()
            return 0
        lax.fori_loop(0, NCHUNK, _drain, 0)

        ninf = jnp.full((LANES,), NEG_INF, jnp.float32)
        zero_i = jnp.zeros((LANES,), jnp.int32)

        def _insert(chain, s, iv):
            m1, m2, m3, i1, i2, i3 = chain
            g1 = s > m1
            n1 = jnp.where(g1, s, m1)
            d1 = jnp.where(g1, m1, s)
            j1 = jnp.where(g1, iv, i1)
            e1 = jnp.where(g1, i1, iv)

            g2 = d1 > m2
            n2 = jnp.where(g2, d1, m2)
            d2 = jnp.where(g2, m2, d1)
            j2 = jnp.where(g2, e1, i2)
            e2 = jnp.where(g2, i2, e1)

            g3 = d2 > m3
            n3 = jnp.where(g3, d2, m3)
            j3 = jnp.where(g3, e2, i3)
            return (n1, n2, n3, j1, j2, j3)

        def _tour(c, carry):
            ca, cb = carry
            l0 = c * CHUNK
            for k in range(VPC):
                o = k * LANES
                s = loss_v[pl.ds(l0 + o, LANES)] - gat_v[pl.ds(l0 + o, LANES)]
                mut = (aa_v[pl.ds(l0 + o, LANES)] << 19) != wt_v[pl.ds(l0 + o, LANES)]
                s = jnp.where(mut, s, ninf)
                iv = l0 + o + lane
                if k % 2 == 0:
                    ca = _insert(ca, s, iv)
                else:
                    cb = _insert(cb, s, iv)
            return ca, cb

        chain0 = (ninf, ninf, ninf, zero_i, zero_i, zero_i)
        ca, cb = lax.fori_loop(0, NCHUNK, _tour, (chain0, chain0))

        (a1, a2, a3, ai1, ai2, ai3) = ca
        (b1, b2, b3, bi1, bi2, bi3) = cb
        c1 = a1 > b3
        c2 = a2 > b2
        c3 = a3 > b1
        m1 = jnp.where(c1, a1, b3)
        m2 = jnp.where(c2, a2, b2)
        m3 = jnp.where(c3, a3, b1)
        i1 = jnp.where(c1, ai1, bi3)
        i2 = jnp.where(c2, ai2, bi2)
        i3 = jnp.where(c3, ai3, bi1)

        s1, j1 = plsc.sort_key_val(m1, i1)
        s2, j2 = plsc.sort_key_val(m2, i2)
        s3, j3 = plsc.sort_key_val(m3, i3)

        r2 = lax.rev(s2, (0,))
        rj2 = lax.rev(j2, (0,))
        c = s1 >= r2
        t = jnp.where(c, s1, r2)
        tj = jnp.where(c, j1, rj2)
        t, tj = plsc.sort_key_val(t, tj)

        r3 = lax.rev(s3, (0,))
        rj3 = lax.rev(j3, (0,))
        c = t >= r3
        u = jnp.where(c, t, r3)
        uj = jnp.where(c, tj, rj3)
        u, uj = plsc.sort_key_val(u, uj)

        keep = (lane >= LANES - 3) & (u > CUTOFF)
        sig = 1.0 / (1.0 + jnp.exp(-jnp.where(keep, u, 0.0)))

        plsc.store_scatter(out_v, [uj], sig, mask=keep)
        pltpu.sync_copy(out_v, out_hbm.at[b])
        plsc.store_scatter(out_v, [uj], jnp.zeros((LANES,), jnp.float32),
                           mask=keep)
        return 0

    lax.fori_loop(0, ROWS_PER_W, do_row, 0)


@jax.jit
def _revor_sc(ent_flat, loss, aa_tensor, aa_wt):
    mesh = plsc.VectorSubcoreMesh(core_axis_name="c", subcore_axis_name="s")
    f = pl.kernel(
        _tec_kernel,
        mesh=mesh,
        out_type=jax.ShapeDtypeStruct((B, L), jnp.float32),
        scratch_types=[
            pltpu.VMEM((L,), jnp.int32),
            pltpu.VMEM((L,), jnp.float32),
            pltpu.VMEM((L,), jnp.int32),
            pltpu.VMEM((L,), jnp.int32),
            pltpu.VMEM((L,), jnp.float32),
            pltpu.VMEM((L,), jnp.float32),
            pltpu.SemaphoreType.DMA,
            pltpu.SemaphoreType.DMA,
        ],
        compiler_params=pltpu.CompilerParams(needs_layout_passes=False),
    )
    return f(ent_flat, loss, aa_tensor, aa_wt)


def kernel(entropy, loss, aa_tensor, aa_wt, max_step):
    del max_step
    ent_nat = (entropy.reshape(8, 8, 64, 128, V)
               .transpose(4, 0, 2, 1, 3)
               .reshape(B * L * V))
    return _revor_sc(ent_nat, loss, aa_tensor, aa_wt)


# speedup vs baseline: 6.7347x; 1.1023x over previous
import jax
import jax.numpy as jnp
from jax import lax
from jax.experimental import pallas as pl
from jax.experimental.pallas import tpu as pltpu
from jax.experimental.pallas import tpu_sc as plsc

B, L, V = 64, 8192, 21
CUTOFF = 0.1
NEG_INF = float("-inf")
LANES = 16
VECS = L // LANES
ROWS_PER_W = 2
CHUNK = 128
NCHUNK = L // CHUNK
VPC = CHUNK // LANES


def _tec_kernel(ent_hbm, loss_hbm, aa_hbm, wt_hbm, out_hbm,
                wt_v, loss0_v, aa0_v, idx0_v, gat0_v,
                loss1_v, aa1_v, idx1_v, gat1_v, out_v,
                sem0, sem1, gsem0, gsem1):
    nc = 2
    wid = lax.axis_index("s") * nc + lax.axis_index("c")
    lane = lax.iota(jnp.int32, LANES)
    b0 = wid * ROWS_PER_W

    cl0 = pltpu.make_async_copy(loss_hbm.at[b0], loss0_v, sem0)
    ca0 = pltpu.make_async_copy(aa_hbm.at[b0], aa0_v, sem0)
    cl1 = pltpu.make_async_copy(loss_hbm.at[b0 + 1], loss1_v, sem1)
    ca1 = pltpu.make_async_copy(aa_hbm.at[b0 + 1], aa1_v, sem1)
    cl0.start()
    ca0.start()
    cl1.start()
    ca1.start()
    pltpu.sync_copy(wt_hbm, wt_v)

    def _zero(j, _):
        out_v[pl.ds(j * LANES, LANES)] = jnp.zeros((LANES,), jnp.float32)
        wt_v[pl.ds(j * LANES, LANES)] = wt_v[pl.ds(j * LANES, LANES)] << 19
        return 0
    lax.fori_loop(0, VECS, _zero, 0)

    def _chunk_cp(c, idx_v, gat_v, gsem):
        return pltpu.make_async_copy(
            ent_hbm.at[idx_v.at[pl.ds(c * CHUNK, CHUNK)]],
            gat_v.at[pl.ds(c * CHUNK, CHUNK)], gsem)

    def _build_and_fire(b, idx_v, gat_v, gsem):
        base = (b // 8) * 65536 + (b % 8) * 128

        def _mkidx(c, cur):
            l0 = c * CHUNK
            for k in range(VPC):
                wts = wt_v[pl.ds(l0 + k * LANES, LANES)]
                idx_v[pl.ds(l0 + k * LANES, LANES)] = cur + (k * LANES) + wts
            _chunk_cp(c, idx_v, gat_v, gsem).start()
            return cur + 1024
        lax.fori_loop(0, NCHUNK, _mkidx, base + lane)

    def _drain(idx_v, gat_v, gsem):
        def _d(c, _):
            _chunk_cp(c, idx_v, gat_v, gsem).wait()
            return 0
        lax.fori_loop(0, NCHUNK, _d, 0)

    ninf = jnp.full((LANES,), NEG_INF, jnp.float32)
    zero_i = jnp.zeros((LANES,), jnp.int32)

    def _insert(chain, s, iv):
        m1, m2, m3, i1, i2, i3 = chain
        g1 = s > m1
        n1 = jnp.where(g1, s, m1)
        d1 = jnp.where(g1, m1, s)
        j1 = jnp.where(g1, iv, i1)
        e1 = jnp.where(g1, i1, iv)
        g2 = d1 > m2
        n2 = jnp.where(g2, d1, m2)
        d2 = jnp.where(g2, m2, d1)
        j2 = jnp.where(g2, e1, i2)
        e2 = jnp.where(g2, i2, e1)
        g3 = d2 > m3
        n3 = jnp.where(g3, d2, m3)
        j3 = jnp.where(g3, e2, i3)
        return (n1, n2, n3, j1, j2, j3)

    def _row_compute(b, loss_v, aa_v, gat_v):
        def _tour(c, carry):
            ca, cb = carry
            l0 = c * CHUNK
            for k in range(VPC):
                o = k * LANES
                s = loss_v[pl.ds(l0 + o, LANES)] - gat_v[pl.ds(l0 + o, LANES)]
                mut = (aa_v[pl.ds(l0 + o, LANES)] << 19) != wt_v[pl.ds(l0 + o, LANES)]
                s = jnp.where(mut, s, ninf)
                iv = l0 + o + lane
                if k % 2 == 0:
                    ca = _insert(ca, s, iv)
                else:
                    cb = _insert(cb, s, iv)
            return ca, cb

        chain0 = (ninf, ninf, ninf, zero_i, zero_i, zero_i)
        ca, cb = lax.fori_loop(0, NCHUNK, _tour, (chain0, chain0))

        (a1, a2, a3, ai1, ai2, ai3) = ca
        (q1, q2, q3, qi1, qi2, qi3) = cb
        c1 = a1 > q3
        c2 = a2 > q2
        c3 = a3 > q1
        m1 = jnp.where(c1, a1, q3)
        m2 = jnp.where(c2, a2, q2)
        m3 = jnp.where(c3, a3, q1)
        i1 = jnp.where(c1, ai1, qi3)
        i2 = jnp.where(c2, ai2, qi2)
        i3 = jnp.where(c3, ai3, qi1)

        s1, j1 = plsc.sort_key_val(m1, i1)
        s2, j2 = plsc.sort_key_val(m2, i2)
        s3, j3 = plsc.sort_key_val(m3, i3)

        r2 = lax.rev(s2, (0,))
        rj2 = lax.rev(j2, (0,))
        c = s1 >= r2
        t = jnp.where(c, s1, r2)
        tj = jnp.where(c, j1, rj2)
        t, tj = plsc.sort_key_val(t, tj)

        r3 = lax.rev(s3, (0,))
        rj3 = lax.rev(j3, (0,))
        c = t >= r3
        u = jnp.where(c, t, r3)
        uj = jnp.where(c, tj, rj3)
        u, uj = plsc.sort_key_val(u, uj)

        keep = (lane >= LANES - 3) & (u > CUTOFF)
        sig = 1.0 / (1.0 + jnp.exp(-jnp.where(keep, u, 0.0)))

        plsc.store_scatter(out_v, [uj], sig, mask=keep)
        pltpu.sync_copy(out_v, out_hbm.at[b])
        plsc.store_scatter(out_v, [uj], jnp.zeros((LANES,), jnp.float32),
                           mask=keep)

    _build_and_fire(b0, idx0_v, gat0_v, gsem0)
    _build_and_fire(b0 + 1, idx1_v, gat1_v, gsem1)

    _drain(idx0_v, gat0_v, gsem0)
    cl0.wait()
    ca0.wait()
    _row_compute(b0, loss0_v, aa0_v, gat0_v)

    _drain(idx1_v, gat1_v, gsem1)
    cl1.wait()
    ca1.wait()
    _row_compute(b0 + 1, loss1_v, aa1_v, gat1_v)


@jax.jit
def _revor_sc(ent_nat, loss, aa_tensor, aa_wt):
    mesh = plsc.VectorSubcoreMesh(core_axis_name="c", subcore_axis_name="s")
    f = pl.kernel(
        _tec_kernel,
        mesh=mesh,
        out_type=jax.ShapeDtypeStruct((B, L), jnp.float32),
        scratch_types=[
            pltpu.VMEM((L,), jnp.int32),
            pltpu.VMEM((L,), jnp.float32),
            pltpu.VMEM((L,), jnp.int32),
            pltpu.VMEM((L,), jnp.int32),
            pltpu.VMEM((L,), jnp.float32),
            pltpu.VMEM((L,), jnp.float32),
            pltpu.VMEM((L,), jnp.int32),
            pltpu.VMEM((L,), jnp.int32),
            pltpu.VMEM((L,), jnp.float32),
            pltpu.VMEM((L,), jnp.float32),
            pltpu.SemaphoreType.DMA,
            pltpu.SemaphoreType.DMA,
            pltpu.SemaphoreType.DMA,
            pltpu.SemaphoreType.DMA,
        ],
        compiler_params=pltpu.CompilerParams(needs_layout_passes=False),
    )
    return f(ent_nat, loss, aa_tensor, aa_wt)


def kernel(entropy, loss, aa_tensor, aa_wt, max_step):
    del max_step
    ent_nat = (entropy.reshape(8, 8, 64, 128, V)
               .transpose(4, 0, 2, 1, 3)
               .reshape(B * L * V))
    return _revor_sc(ent_nat, loss, aa_tensor, aa_wt)
